# baseline (speedup 1.0000x reference)
import functools

import jax
import jax.numpy as jnp
from jax import lax
from jax.experimental import pallas as pl
from jax.experimental.pallas import tpu as pltpu
from jax.experimental.pallas import tpu_sc as plsc

_NC = 2
_NS = 16
_NW = _NC * _NS
_L = 16


def _sc_body(rows_per_w, ch, d, x_hbm, emb_hbm, idx_hbm, out_hbm,
             idx_v, row_v, buf):
    wid = lax.axis_index("s") * _NC + lax.axis_index("c")
    base = wid * rows_per_w

    pltpu.sync_copy(idx_hbm, idx_v)
    i = idx_v[...][0]
    pltpu.sync_copy(emb_hbm.at[i], row_v)

    nch = rows_per_w // ch
    nj = d // _L

    def chunk(g, carry):
        lo = base + g * ch
        pltpu.sync_copy(x_hbm.at[pl.ds(lo, ch)], buf)

        def rowfn(r, c2):
            for j in range(nj):
                plsc.addupdate(buf.at[r, pl.ds(j * _L, _L)],
                               row_v[pl.ds(j * _L, _L)])
            return c2

        lax.fori_loop(0, ch, rowfn, 0)
        pltpu.sync_copy(buf, out_hbm.at[pl.ds(lo, ch)])
        return carry

    lax.fori_loop(0, nch, chunk, 0)


def kernel(x, type_emb, index):
    B, S, D = x.shape
    N = B * S
    assert D % _L == 0
    xf = x.reshape(N, D)
    idx = jnp.broadcast_to(jnp.asarray(index, jnp.int32), (_L,))

    rows_per_w = N // _NW
    ch = 64

    mesh = plsc.VectorSubcoreMesh(core_axis_name="c", subcore_axis_name="s")
    body = functools.partial(_sc_body, rows_per_w, ch, D)
    out = pl.kernel(
        body,
        out_type=jax.ShapeDtypeStruct((N, D), x.dtype),
        mesh=mesh,
        scratch_types=[
            pltpu.VMEM((_L,), jnp.int32),
            pltpu.VMEM((D,), jnp.float32),
            pltpu.VMEM((ch, D), jnp.float32),
        ],
    )(xf, type_emb, idx)
    return out.reshape(B, S, D)


# speedup vs baseline: 2.9685x; 2.9685x over previous
import jax
import jax.numpy as jnp
from jax import lax
from jax.experimental import pallas as pl
from jax.experimental.pallas import tpu as pltpu
from jax.experimental.pallas import tpu_sc as plsc

_NC = 2
_NS = 16
_L = 16


def _sc_gather_body(emb_hbm, idx_hbm, row_hbm, idx_v, row_v):
    wid = lax.axis_index("s") * _NC + lax.axis_index("c")

    @pl.when(wid == 0)
    def _():
        pltpu.sync_copy(idx_hbm, idx_v)
        i = idx_v[...][0]
        pltpu.sync_copy(emb_hbm.at[i], row_v)
        pltpu.sync_copy(row_v, row_hbm.at[0])


def _tc_add_body(x_ref, row_ref, o_ref):
    o_ref[...] = x_ref[...] + row_ref[...]


def kernel(x, type_emb, index):
    B, S, D = x.shape
    N = B * S
    xf = x.reshape(N, D)
    idx = jnp.broadcast_to(jnp.asarray(index, jnp.int32), (_L,))

    mesh = plsc.VectorSubcoreMesh(core_axis_name="c", subcore_axis_name="s")
    row = pl.kernel(
        _sc_gather_body,
        out_type=jax.ShapeDtypeStruct((1, D), jnp.float32),
        mesh=mesh,
        scratch_types=[
            pltpu.VMEM((_L,), jnp.int32),
            pltpu.VMEM((D,), jnp.float32),
        ],
    )(type_emb, idx)

    BM = 2048
    out = pl.pallas_call(
        _tc_add_body,
        grid=(N // BM,),
        in_specs=[
            pl.BlockSpec((BM, D), lambda i: (i, 0)),
            pl.BlockSpec((1, D), lambda i: (0, 0)),
        ],
        out_specs=pl.BlockSpec((BM, D), lambda i: (i, 0)),
        out_shape=jax.ShapeDtypeStruct((N, D), x.dtype),
    )(xf, row)
    return out.reshape(B, S, D)


# speedup vs baseline: 4.3532x; 1.4665x over previous
import jax
import jax.numpy as jnp
from jax.experimental import pallas as pl
from jax.experimental.pallas import tpu as pltpu


def _body(idx_ref, x_ref, emb_ref, o_ref):
    i = idx_ref[0]
    row = emb_ref[pl.ds(i, 1), :]
    o_ref[...] = x_ref[...] + row


def kernel(x, type_emb, index):
    B, S, D = x.shape
    N = B * S
    xf = x.reshape(N, D)
    idx = jnp.asarray(index, jnp.int32).reshape(1)

    BM = 2048
    grid = (N // BM,)

    out = pl.pallas_call(
        _body,
        grid_spec=pltpu.PrefetchScalarGridSpec(
            num_scalar_prefetch=1,
            grid=grid,
            in_specs=[
                pl.BlockSpec((BM, D), lambda i, s: (i, 0)),
                pl.BlockSpec((2, D), lambda i, s: (0, 0)),
            ],
            out_specs=pl.BlockSpec((BM, D), lambda i, s: (i, 0)),
        ),
        out_shape=jax.ShapeDtypeStruct((N, D), x.dtype),
    )(idx, xf, type_emb)
    return out.reshape(B, S, D)
